# baseline (speedup 1.0000x reference)
import functools

import jax
import jax.numpy as jnp
from jax import lax
from jax.experimental import pallas as pl
from jax.experimental.pallas import tpu as pltpu
from jax.experimental.pallas import tpu_sc as plsc

_EPS = 1e-12

_NUM_CORES = 2
_NUM_SUBCORES = 16
_NUM_WORKERS = _NUM_CORES * _NUM_SUBCORES
_CHUNK = 64


def _sc_gather(word_table, flat_ids):
    n = flat_ids.shape[0]
    d = word_table.shape[1]
    b_per_w = n // _NUM_WORKERS
    n_chunks = b_per_w // _CHUNK
    assert n_chunks % 2 == 0 and n_chunks >= 4
    mesh = plsc.VectorSubcoreMesh(core_axis_name="c", subcore_axis_name="s")

    @functools.partial(
        pl.kernel,
        mesh=mesh,
        out_type=jax.ShapeDtypeStruct((n, d), jnp.float32),
        scratch_types=[
            pltpu.VMEM((b_per_w,), jnp.int32),
            pltpu.VMEM((_CHUNK, d), jnp.float32),
            pltpu.VMEM((_CHUNK, d), jnp.float32),
            pltpu.SemaphoreType.DMA,
            pltpu.SemaphoreType.DMA,
        ],
    )
    def gather_kernel(table_hbm, idx_hbm, out_hbm, idx_v, rows0, rows1, sem0, sem1):
        wid = lax.axis_index("s") * _NUM_CORES + lax.axis_index("c")
        base = wid * b_per_w
        pltpu.sync_copy(idx_hbm.at[pl.ds(base, b_per_w)], idx_v)

        def gather_start(c, buf, sem):
            off = pl.multiple_of(c * _CHUNK, _CHUNK)
            return pltpu.make_async_copy(
                table_hbm.at[idx_v.at[pl.ds(off, _CHUNK)]], buf, sem
            )

        def write_out(c, buf):
            off = pl.multiple_of(c * _CHUNK, _CHUNK)
            pltpu.sync_copy(buf, out_hbm.at[pl.ds(base + off, _CHUNK)])

        gather_start(0, rows0, sem0).start()
        gather_start(1, rows1, sem1).start()

        @pl.loop(0, n_chunks - 2, step=2)
        def _(c):
            gather_start(c, rows0, sem0).wait()
            write_out(c, rows0)
            gather_start(c + 2, rows0, sem0).start()
            gather_start(c + 1, rows1, sem1).wait()
            write_out(c + 1, rows1)
            gather_start(c + 3, rows1, sem1).start()

        gather_start(n_chunks - 2, rows0, sem0).wait()
        write_out(n_chunks - 2, rows0)
        gather_start(n_chunks - 1, rows1, sem1).wait()
        write_out(n_chunks - 1, rows1)

    return gather_kernel(word_table, flat_ids)


_BB = 4


def _ln_body(x_ref, pos_ref, g_ref, beta_ref, o_ref):
    x = x_ref[...] + pos_ref[...][None, :, :]
    mean = jnp.mean(x, axis=-1, keepdims=True)
    xc = x - mean
    var = jnp.mean(xc * xc, axis=-1, keepdims=True)
    inv = lax.rsqrt(var + _EPS)
    o_ref[...] = xc * inv * g_ref[...] + beta_ref[...]


def _tc_add_ln_slice(full_b, block_base, prev_out, gathered, pos_table, ln_gamma, ln_beta):
    bs, l, d = gathered.shape

    data_specs = [
        pl.BlockSpec((_BB, l, d), lambda i: (i, 0, 0)),
        pl.BlockSpec((l, d), lambda i: (0, 0)),
        pl.BlockSpec((d,), lambda i: (0,)),
        pl.BlockSpec((d,), lambda i: (0,)),
    ]
    out_spec = pl.BlockSpec((_BB, l, d), lambda i: (block_base + i, 0, 0))
    out_shape = jax.ShapeDtypeStruct((full_b, l, d), jnp.float32)
    args = (gathered, pos_table, ln_gamma, ln_beta)

    if prev_out is None:
        body = _ln_body
        in_specs = data_specs
        aliases = {}
    else:
        def body(big_ref, *refs):
            del big_ref
            _ln_body(*refs)

        in_specs = [pl.BlockSpec(memory_space=pl.ANY)] + data_specs
        aliases = {0: 0}
        args = (prev_out,) + args

    return pl.pallas_call(
        body,
        grid=(bs // _BB,),
        in_specs=in_specs,
        out_specs=out_spec,
        out_shape=out_shape,
        input_output_aliases=aliases,
    )(*args)


def kernel(input_ids, word_table, pos_table, ln_gamma, ln_beta):
    b, l = input_ids.shape
    d = word_table.shape[1]
    n_slices = 4
    bs = b // n_slices
    flat_ids = input_ids.reshape(-1).astype(jnp.int32)

    gathered = [
        _sc_gather(word_table, flat_ids[i * bs * l:(i + 1) * bs * l])
        for i in range(n_slices)
    ]
    out = None
    for i in range(n_slices):
        out = _tc_add_ln_slice(
            b, i * (bs // _BB), out, gathered[i].reshape(bs, l, d),
            pos_table, ln_gamma, ln_beta,
        )
    return out
